# baseline (speedup 1.0000x reference)
import functools

import jax
import jax.numpy as jnp
from jax import lax
from jax.experimental import pallas as pl
from jax.experimental.pallas import tpu as pltpu
from jax.experimental.pallas import tpu_sc as plsc

_B, _S, _D = 2, 2048, 1024
_I = 4096
_E = 8
_T = _B * _S
_A = 2 * _T
_BT = 512
_NB = _A // _BT + _E
_P = _NB * _BT
_NW = 32
_TPW = _T // _NW
_CH = 32
_NCH = _TPW // _CH

_SQRT1_2 = 0.7071067811865476


def _router_body(x_ref, wr_ref, br_ref, slots_ref, gates_ref, bexp_ref,
                 nb_ref, incol_ref):
    logits = jnp.dot(x_ref[...], wr_ref[...],
                     preferred_element_type=jnp.float32) + br_ref[...]
    eidx = lax.broadcasted_iota(jnp.int32, (_T, _E), 1)
    m1 = jnp.max(logits, axis=-1, keepdims=True)
    a1 = jnp.min(jnp.where(logits == m1, eidx, _E), axis=-1, keepdims=True)
    masked = jnp.where(eidx == a1, -1e30, logits)
    m2 = jnp.max(masked, axis=-1, keepdims=True)
    a2 = jnp.min(jnp.where(masked == m2, eidx, _E), axis=-1, keepdims=True)
    e2 = jnp.exp(m2 - m1)
    denom = 1.0 + e2
    g1 = 1.0 / denom
    g2 = e2 / denom

    h1 = (eidx == a1).astype(jnp.float32)
    h2 = (eidx == a2).astype(jnp.float32)
    H = jnp.concatenate([h1, h2], axis=0)

    running = jnp.zeros((128, _E), jnp.float32)
    for u in range(64):
        incol_ref[pl.ds(u * 128, 128), :] = running
        running = running + H[u * 128:(u + 1) * 128, :]
    colcnt = running

    qv = lax.rem(lax.broadcasted_iota(jnp.int32, (_A, 128), 0), 128)
    cv = lax.broadcasted_iota(jnp.int32, (_A, 128), 1)
    Lv = (cv < qv).astype(jnp.float32)
    prefv = jnp.dot(Lv, colcnt, preferred_element_type=jnp.float32)
    rank_pe = prefv + incol_ref[...]

    counts = jnp.sum(colcnt, axis=0, keepdims=True)
    counts_i = counts.astype(jnp.int32)
    padded_i = ((counts_i + _BT - 1) // _BT) * _BT
    padded_f = padded_i.astype(jnp.float32)

    er = lax.broadcasted_iota(jnp.int32, (_E, _E), 0)
    ec = lax.broadcasted_iota(jnp.int32, (_E, _E), 1)
    pad_mat = jnp.broadcast_to(padded_f.reshape(_E, 1), (_E, _E))
    off = jnp.sum(jnp.where(er < ec, pad_mat, 0.0), axis=0,
                  keepdims=True)

    slot_f = jnp.sum(H * (rank_pe + off), axis=1, keepdims=True)
    slots_ref[...] = slot_f.astype(jnp.int32)
    G = jnp.concatenate([g1, g2], axis=0)
    gates_ref[...] = jnp.broadcast_to(G, (_A, 16))

    off_i = off.astype(jnp.int32)
    nb = jnp.sum(padded_i, axis=1, keepdims=True) // _BT
    bstart = jnp.broadcast_to(off_i.reshape(_E, 1), (_E, _NB))
    bend = bstart + jnp.broadcast_to(padded_i.reshape(_E, 1), (_E, _NB))
    bpos = lax.broadcasted_iota(jnp.int32, (_E, _NB), 1) * _BT
    erow = lax.broadcasted_iota(jnp.int32, (_E, _NB), 0)
    bexp = jnp.sum(jnp.where((bpos >= bstart) & (bpos < bend), erow, 0),
                   axis=0, keepdims=True)
    erng = lax.broadcasted_iota(jnp.int32, (1, _E), 1)
    last_e = jnp.max(jnp.where(padded_i > 0, erng, 0), axis=1,
                     keepdims=True)
    binb = lax.broadcasted_iota(jnp.int32, (1, _NB), 1)
    bexp_ref[...] = jnp.where(binb < nb, bexp, last_e)
    nb_ref[...] = nb


def _router(x_flat, Wr, br):
    return pl.pallas_call(
        _router_body,
        out_shape=(
            jax.ShapeDtypeStruct((_A, 1), jnp.int32),
            jax.ShapeDtypeStruct((_A, 16), jnp.float32),
            jax.ShapeDtypeStruct((1, _NB), jnp.int32),
            jax.ShapeDtypeStruct((1, 1), jnp.int32),
        ),
        scratch_shapes=[pltpu.VMEM((_A, _E), jnp.float32)],
    )(x_flat, Wr, br)


@functools.lru_cache(maxsize=None)
def _sc_scatter_kernel():
    mesh = plsc.VectorSubcoreMesh(core_axis_name="c", subcore_axis_name="s")

    @functools.partial(
        pl.kernel,
        mesh=mesh,
        out_type=jax.ShapeDtypeStruct((_P, _D), jnp.float32),
        scratch_types=[
            pltpu.VMEM((_CH, _D), jnp.float32),
            pltpu.VMEM((_CH, _D), jnp.float32),
            pltpu.VMEM((_CH,), jnp.int32),
            pltpu.VMEM((_CH,), jnp.int32),
            pltpu.VMEM((_CH,), jnp.int32),
            pltpu.VMEM((_CH,), jnp.int32),
            pltpu.SemaphoreType.DMA,
        ],
    )
    def _sc_scatter(x_hbm, slots_hbm, xs_hbm, rows_a, rows_b,
                    idx0a, idx1a, idx0b, idx1b, sem):
        w = lax.axis_index("s") * 2 + lax.axis_index("c")
        rows = (rows_a, rows_b)
        idx0 = (idx0a, idx0b)
        idx1 = (idx1a, idx1b)
        cps = []
        for ci in range(_NCH):
            par = ci & 1
            if ci >= 2:
                cps[ci - 2][0].wait()
                cps[ci - 2][1].wait()
            base = w * _TPW + ci * _CH
            pltpu.sync_copy(x_hbm.at[pl.ds(base, _CH)], rows[par])
            pltpu.sync_copy(slots_hbm.at[0, w, ci], idx0[par])
            pltpu.sync_copy(slots_hbm.at[1, w, ci], idx1[par])
            cps.append((
                pltpu.async_copy(rows[par], xs_hbm.at[idx0[par]], sem),
                pltpu.async_copy(rows[par], xs_hbm.at[idx1[par]], sem),
            ))
        for pair in cps[-2:]:
            pair[0].wait()
            pair[1].wait()

    return _sc_scatter


def _ffn_body(bexp_ref, nb_ref, xs_ref, w1_ref, b1_ref, w2_ref, b2_ref,
              ys_ref):
    b = pl.program_id(0)

    @pl.when(b < nb_ref[0])
    def _():
        xb = xs_ref[...].astype(jnp.bfloat16)
        h = jnp.dot(xb, w1_ref[0], preferred_element_type=jnp.float32)
        h = h + b1_ref[0, 0]
        h = 0.5 * h * (1.0 + lax.erf(h * _SQRT1_2))
        y = jnp.dot(h.astype(jnp.bfloat16), w2_ref[0],
                    preferred_element_type=jnp.float32)
        ys_ref[...] = y + b2_ref[0, 0]


def _ffn(bexp, nb, xs, W1_bf, b1, W2_bf, b2):
    grid_spec = pltpu.PrefetchScalarGridSpec(
        num_scalar_prefetch=2,
        grid=(_NB,),
        in_specs=[
            pl.BlockSpec((_BT, _D), lambda b, be, nb: (b, 0)),
            pl.BlockSpec((1, _D, _I), lambda b, be, nb: (be[b], 0, 0)),
            pl.BlockSpec((1, 1, _I), lambda b, be, nb: (be[b], 0, 0)),
            pl.BlockSpec((1, _I, _D), lambda b, be, nb: (be[b], 0, 0)),
            pl.BlockSpec((1, 1, _D), lambda b, be, nb: (be[b], 0, 0)),
        ],
        out_specs=pl.BlockSpec((_BT, _D), lambda b, be, nb: (b, 0)),
    )
    return pl.pallas_call(
        _ffn_body,
        grid_spec=grid_spec,
        out_shape=jax.ShapeDtypeStruct((_P, _D), jnp.float32),
    )(bexp, nb, xs, W1_bf, b1.reshape(_E, 1, _I), W2_bf,
      b2.reshape(_E, 1, _D))


_CC = 16
_CN = _TPW // _CC


@functools.lru_cache(maxsize=None)
def _sc_combine_kernel():
    mesh = plsc.VectorSubcoreMesh(core_axis_name="c", subcore_axis_name="s")

    @functools.partial(
        pl.kernel,
        mesh=mesh,
        out_type=jax.ShapeDtypeStruct((_T, _D), jnp.float32),
        scratch_types=[
            pltpu.VMEM((_CC, _D), jnp.float32),
            pltpu.VMEM((_CC, _D), jnp.float32),
            pltpu.VMEM((_CC, _D), jnp.float32),
            pltpu.VMEM((_CC, _D), jnp.float32),
            pltpu.VMEM((_CC, _D), jnp.float32),
            pltpu.VMEM((_CC, _D), jnp.float32),
            pltpu.VMEM((_CC,), jnp.int32),
            pltpu.VMEM((_CC,), jnp.int32),
            pltpu.VMEM((_CC,), jnp.int32),
            pltpu.VMEM((_CC,), jnp.int32),
            pltpu.VMEM((_CC, 16), jnp.float32),
            pltpu.VMEM((_CC, 16), jnp.float32),
            pltpu.VMEM((_CC, 16), jnp.float32),
            pltpu.VMEM((_CC, 16), jnp.float32),
            pltpu.SemaphoreType.DMA,
            pltpu.SemaphoreType.DMA,
            pltpu.SemaphoreType.DMA,
        ],
    )
    def _sc_combine(ys_hbm, slots_hbm, gates_hbm, out_hbm,
                    r0a, r0b, r1a, r1b, oba, obb,
                    idx0a, idx0b, idx1a, idx1b,
                    g0a, g0b, g1a, g1b, sem0, sem1, semo):
        w = lax.axis_index("s") * 2 + lax.axis_index("c")
        r0 = (r0a, r0b)
        r1 = (r1a, r1b)
        ob = (oba, obb)
        idx0 = (idx0a, idx0b)
        idx1 = (idx1a, idx1b)
        g0 = (g0a, g0b)
        g1 = (g1a, g1b)

        def fire(ci):
            par = ci & 1
            pltpu.sync_copy(slots_hbm.at[0, w, ci], idx0[par])
            pltpu.sync_copy(slots_hbm.at[1, w, ci], idx1[par])
            pltpu.sync_copy(gates_hbm.at[0, w, ci], g0[par])
            pltpu.sync_copy(gates_hbm.at[1, w, ci], g1[par])
            return (pltpu.async_copy(ys_hbm.at[idx0[par]], r0[par], sem0),
                    pltpu.async_copy(ys_hbm.at[idx1[par]], r1[par], sem1))

        cps = fire(0)
        ocp = None
        for ci in range(_CN):
            par = ci & 1
            cps[0].wait()
            cps[1].wait()
            if ci + 1 < _CN:
                cps = fire(ci + 1)
            if ocp is not None:
                ocp.wait()

            def body(i, carry):
                gv0 = g0[par][i]
                gv1 = g1[par][i]
                for j in range(_D // 16):
                    sl = pl.ds(j * 16, 16)
                    ob[par][i, sl] = r0[par][i, sl] * gv0 + r1[par][i, sl] * gv1
                return carry

            lax.fori_loop(0, _CC, body, 0)
            base = w * _TPW + ci * _CC
            ocp = pltpu.async_copy(ob[par], out_hbm.at[pl.ds(base, _CC)],
                                   semo)
        ocp.wait()

    return _sc_combine


def kernel(x, Wr, br, W1, b1, W2, b2):
    x_flat = x.reshape(_T, _D)
    slots, gates, bexp, nb = _router(x_flat, Wr, br)
    xs = _sc_scatter_kernel()(x_flat, slots.reshape(2, _NW, _NCH, _CH))
    ys = _ffn(bexp.reshape(_NB), nb.reshape(1), xs,
              W1.astype(jnp.bfloat16), b1, W2.astype(jnp.bfloat16), b2)
    out = _sc_combine_kernel()(ys, slots.reshape(2, _NW, _CN, _CC),
                               gates.reshape(2, _NW, _CN, _CC, 16))
    return out.reshape(_B, _S, _D)


# speedup vs baseline: 1.0133x; 1.0133x over previous
import functools

import jax
import jax.numpy as jnp
from jax import lax
from jax.experimental import pallas as pl
from jax.experimental.pallas import tpu as pltpu
from jax.experimental.pallas import tpu_sc as plsc

_B, _S, _D = 2, 2048, 1024
_I = 4096
_E = 8
_T = _B * _S
_A = 2 * _T
_BT = 384
_NB = -(-_A // _BT) + _E
_P = _NB * _BT
_NW = 32
_TPW = _T // _NW
_CH = 32
_NCH = _TPW // _CH

_SQRT1_2 = 0.7071067811865476


def _router_body(x_ref, wr_ref, br_ref, slots_ref, gates_ref, bexp_ref,
                 nb_ref, incol_ref):
    logits = jnp.dot(x_ref[...], wr_ref[...],
                     preferred_element_type=jnp.float32) + br_ref[...]
    eidx = lax.broadcasted_iota(jnp.int32, (_T, _E), 1)
    m1 = jnp.max(logits, axis=-1, keepdims=True)
    a1 = jnp.min(jnp.where(logits == m1, eidx, _E), axis=-1, keepdims=True)
    masked = jnp.where(eidx == a1, -1e30, logits)
    m2 = jnp.max(masked, axis=-1, keepdims=True)
    a2 = jnp.min(jnp.where(masked == m2, eidx, _E), axis=-1, keepdims=True)
    e2 = jnp.exp(m2 - m1)
    denom = 1.0 + e2
    g1 = 1.0 / denom
    g2 = e2 / denom

    h1 = (eidx == a1).astype(jnp.float32)
    h2 = (eidx == a2).astype(jnp.float32)
    H = jnp.concatenate([h1, h2], axis=0)

    running = jnp.zeros((128, _E), jnp.float32)
    for u in range(64):
        incol_ref[pl.ds(u * 128, 128), :] = running
        running = running + H[u * 128:(u + 1) * 128, :]
    colcnt = running

    qv = lax.rem(lax.broadcasted_iota(jnp.int32, (_A, 128), 0), 128)
    cv = lax.broadcasted_iota(jnp.int32, (_A, 128), 1)
    Lv = (cv < qv).astype(jnp.float32)
    prefv = jnp.dot(Lv, colcnt, preferred_element_type=jnp.float32)
    rank_pe = prefv + incol_ref[...]

    counts = jnp.sum(colcnt, axis=0, keepdims=True)
    counts_i = counts.astype(jnp.int32)
    padded_i = ((counts_i + _BT - 1) // _BT) * _BT
    padded_f = padded_i.astype(jnp.float32)

    er = lax.broadcasted_iota(jnp.int32, (_E, _E), 0)
    ec = lax.broadcasted_iota(jnp.int32, (_E, _E), 1)
    pad_mat = jnp.broadcast_to(padded_f.reshape(_E, 1), (_E, _E))
    off = jnp.sum(jnp.where(er < ec, pad_mat, 0.0), axis=0,
                  keepdims=True)

    slot_f = jnp.sum(H * (rank_pe + off), axis=1, keepdims=True)
    slots_ref[...] = slot_f.astype(jnp.int32)
    G = jnp.concatenate([g1, g2], axis=0)
    gates_ref[...] = jnp.broadcast_to(G, (_A, 16))

    off_i = off.astype(jnp.int32)
    nb = jnp.sum(padded_i, axis=1, keepdims=True) // _BT
    bstart = jnp.broadcast_to(off_i.reshape(_E, 1), (_E, _NB))
    bend = bstart + jnp.broadcast_to(padded_i.reshape(_E, 1), (_E, _NB))
    bpos = lax.broadcasted_iota(jnp.int32, (_E, _NB), 1) * _BT
    erow = lax.broadcasted_iota(jnp.int32, (_E, _NB), 0)
    bexp = jnp.sum(jnp.where((bpos >= bstart) & (bpos < bend), erow, 0),
                   axis=0, keepdims=True)
    erng = lax.broadcasted_iota(jnp.int32, (1, _E), 1)
    last_e = jnp.max(jnp.where(padded_i > 0, erng, 0), axis=1,
                     keepdims=True)
    binb = lax.broadcasted_iota(jnp.int32, (1, _NB), 1)
    bexp_ref[...] = jnp.where(binb < nb, bexp, last_e)
    nb_ref[...] = nb


def _router(x_flat, Wr, br):
    return pl.pallas_call(
        _router_body,
        out_shape=(
            jax.ShapeDtypeStruct((_A, 1), jnp.int32),
            jax.ShapeDtypeStruct((_A, 16), jnp.float32),
            jax.ShapeDtypeStruct((1, _NB), jnp.int32),
            jax.ShapeDtypeStruct((1, 1), jnp.int32),
        ),
        scratch_shapes=[pltpu.VMEM((_A, _E), jnp.float32)],
    )(x_flat, Wr, br)


@functools.lru_cache(maxsize=None)
def _sc_scatter_kernel():
    mesh = plsc.VectorSubcoreMesh(core_axis_name="c", subcore_axis_name="s")

    @functools.partial(
        pl.kernel,
        mesh=mesh,
        out_type=jax.ShapeDtypeStruct((_P, _D), jnp.float32),
        scratch_types=[
            pltpu.VMEM((_CH, _D), jnp.float32),
            pltpu.VMEM((_CH, _D), jnp.float32),
            pltpu.VMEM((_CH,), jnp.int32),
            pltpu.VMEM((_CH,), jnp.int32),
            pltpu.VMEM((_CH,), jnp.int32),
            pltpu.VMEM((_CH,), jnp.int32),
            pltpu.SemaphoreType.DMA,
        ],
    )
    def _sc_scatter(x_hbm, slots_hbm, xs_hbm, rows_a, rows_b,
                    idx0a, idx1a, idx0b, idx1b, sem):
        w = lax.axis_index("s") * 2 + lax.axis_index("c")
        rows = (rows_a, rows_b)
        idx0 = (idx0a, idx0b)
        idx1 = (idx1a, idx1b)
        cps = []
        for ci in range(_NCH):
            par = ci & 1
            if ci >= 2:
                cps[ci - 2][0].wait()
                cps[ci - 2][1].wait()
            base = w * _TPW + ci * _CH
            pltpu.sync_copy(x_hbm.at[pl.ds(base, _CH)], rows[par])
            pltpu.sync_copy(slots_hbm.at[0, w, ci], idx0[par])
            pltpu.sync_copy(slots_hbm.at[1, w, ci], idx1[par])
            cps.append((
                pltpu.async_copy(rows[par], xs_hbm.at[idx0[par]], sem),
                pltpu.async_copy(rows[par], xs_hbm.at[idx1[par]], sem),
            ))
        for pair in cps[-2:]:
            pair[0].wait()
            pair[1].wait()

    return _sc_scatter


def _ffn_body(bexp_ref, nb_ref, xs_ref, w1_ref, b1_ref, w2_ref, b2_ref,
              ys_ref):
    b = pl.program_id(0)

    @pl.when(b < nb_ref[0])
    def _():
        xb = xs_ref[...].astype(jnp.bfloat16)
        h = jnp.dot(xb, w1_ref[0], preferred_element_type=jnp.float32)
        h = h + b1_ref[0, 0]
        h = 0.5 * h * (1.0 + lax.erf(h * _SQRT1_2))
        y = jnp.dot(h.astype(jnp.bfloat16), w2_ref[0],
                    preferred_element_type=jnp.float32)
        ys_ref[...] = y + b2_ref[0, 0]


def _ffn(bexp, nb, xs, W1_bf, b1, W2_bf, b2):
    grid_spec = pltpu.PrefetchScalarGridSpec(
        num_scalar_prefetch=2,
        grid=(_NB,),
        in_specs=[
            pl.BlockSpec((_BT, _D), lambda b, be, nb: (b, 0)),
            pl.BlockSpec((1, _D, _I), lambda b, be, nb: (be[b], 0, 0)),
            pl.BlockSpec((1, 1, _I), lambda b, be, nb: (be[b], 0, 0)),
            pl.BlockSpec((1, _I, _D), lambda b, be, nb: (be[b], 0, 0)),
            pl.BlockSpec((1, 1, _D), lambda b, be, nb: (be[b], 0, 0)),
        ],
        out_specs=pl.BlockSpec((_BT, _D), lambda b, be, nb: (b, 0)),
    )
    return pl.pallas_call(
        _ffn_body,
        grid_spec=grid_spec,
        out_shape=jax.ShapeDtypeStruct((_P, _D), jnp.float32),
    )(bexp, nb, xs, W1_bf, b1.reshape(_E, 1, _I), W2_bf,
      b2.reshape(_E, 1, _D))


_CC = 16
_CN = _TPW // _CC


@functools.lru_cache(maxsize=None)
def _sc_combine_kernel():
    mesh = plsc.VectorSubcoreMesh(core_axis_name="c", subcore_axis_name="s")

    @functools.partial(
        pl.kernel,
        mesh=mesh,
        out_type=jax.ShapeDtypeStruct((_T, _D), jnp.float32),
        scratch_types=[
            pltpu.VMEM((_CC, _D), jnp.float32),
            pltpu.VMEM((_CC, _D), jnp.float32),
            pltpu.VMEM((_CC, _D), jnp.float32),
            pltpu.VMEM((_CC, _D), jnp.float32),
            pltpu.VMEM((_CC, _D), jnp.float32),
            pltpu.VMEM((_CC, _D), jnp.float32),
            pltpu.VMEM((_CC,), jnp.int32),
            pltpu.VMEM((_CC,), jnp.int32),
            pltpu.VMEM((_CC,), jnp.int32),
            pltpu.VMEM((_CC,), jnp.int32),
            pltpu.VMEM((_CC, 16), jnp.float32),
            pltpu.VMEM((_CC, 16), jnp.float32),
            pltpu.VMEM((_CC, 16), jnp.float32),
            pltpu.VMEM((_CC, 16), jnp.float32),
            pltpu.SemaphoreType.DMA,
            pltpu.SemaphoreType.DMA,
            pltpu.SemaphoreType.DMA,
        ],
    )
    def _sc_combine(ys_hbm, slots_hbm, gates_hbm, out_hbm,
                    r0a, r0b, r1a, r1b, oba, obb,
                    idx0a, idx0b, idx1a, idx1b,
                    g0a, g0b, g1a, g1b, sem0, sem1, semo):
        w = lax.axis_index("s") * 2 + lax.axis_index("c")
        r0 = (r0a, r0b)
        r1 = (r1a, r1b)
        ob = (oba, obb)
        idx0 = (idx0a, idx0b)
        idx1 = (idx1a, idx1b)
        g0 = (g0a, g0b)
        g1 = (g1a, g1b)

        def fire(ci):
            par = ci & 1
            pltpu.sync_copy(slots_hbm.at[0, w, ci], idx0[par])
            pltpu.sync_copy(slots_hbm.at[1, w, ci], idx1[par])
            pltpu.sync_copy(gates_hbm.at[0, w, ci], g0[par])
            pltpu.sync_copy(gates_hbm.at[1, w, ci], g1[par])
            return (pltpu.async_copy(ys_hbm.at[idx0[par]], r0[par], sem0),
                    pltpu.async_copy(ys_hbm.at[idx1[par]], r1[par], sem1))

        cps = fire(0)
        ocp = None
        for ci in range(_CN):
            par = ci & 1
            cps[0].wait()
            cps[1].wait()
            if ci + 1 < _CN:
                cps = fire(ci + 1)
            if ocp is not None:
                ocp.wait()

            def body(i, carry):
                gv0 = g0[par][i]
                gv1 = g1[par][i]
                for j in range(_D // 16):
                    sl = pl.ds(j * 16, 16)
                    ob[par][i, sl] = r0[par][i, sl] * gv0 + r1[par][i, sl] * gv1
                return carry

            lax.fori_loop(0, _CC, body, 0)
            base = w * _TPW + ci * _CC
            ocp = pltpu.async_copy(ob[par], out_hbm.at[pl.ds(base, _CC)],
                                   semo)
        ocp.wait()

    return _sc_combine


def kernel(x, Wr, br, W1, b1, W2, b2):
    x_flat = x.reshape(_T, _D)
    slots, gates, bexp, nb = _router(x_flat, Wr, br)
    xs = _sc_scatter_kernel()(x_flat, slots.reshape(2, _NW, _NCH, _CH))
    ys = _ffn(bexp.reshape(_NB), nb.reshape(1), xs,
              W1.astype(jnp.bfloat16), b1, W2.astype(jnp.bfloat16), b2)
    out = _sc_combine_kernel()(ys, slots.reshape(2, _NW, _CN, _CC),
                               gates.reshape(2, _NW, _CN, _CC, 16))
    return out.reshape(_B, _S, _D)
